# Initial kernel scaffold; baseline (speedup 1.0000x reference)
import jax
import jax.numpy as jnp
from jax.experimental import pallas as pl


def kernel(x, edge_index, edge_attr, emb, W1q, b1q, W1k, b1k, W1v, b1v, W1e, W1s, b1s, W2q, b2q, W2k, b2k, W2v, b2v, W2e, W2s, b2s, Wo, bo):
    raise NotImplementedError("write your pallas kernel here")


# speedup vs baseline: 3.1112x; 3.1112x over previous
import dataclasses
import functools

import jax
import jax.numpy as jnp
import numpy as np
from jax import lax
from jax.experimental import pallas as pl
from jax.experimental.pallas import tpu as pltpu
from jax.experimental.pallas import tpu_sc as plsc

_H = 64
_C = 128
_NTILES = 32
_LANES = 16


def _proj1_body(x_ref, emb_ref, wq, wk, wv, ws, we1, we2, bias_ref,
                q_ref, kv_ref, skip_ref, e1_ref, e2_ref):
    xb = x_ref[...]
    f32 = jnp.float32
    q_ref[...] = jnp.dot(xb, wq[...], preferred_element_type=f32) + bias_ref[0:1, :]
    kv_ref[:, :_H] = jnp.dot(xb, wk[...], preferred_element_type=f32) + bias_ref[1:2, :]
    kv_ref[:, _H:] = jnp.dot(xb, wv[...], preferred_element_type=f32) + bias_ref[2:3, :]
    skip_ref[...] = jnp.dot(xb, ws[...], preferred_element_type=f32) + bias_ref[3:4, :]

    @pl.when(pl.program_id(0) == 0)
    def _():
        e1_ref[...] = jnp.dot(emb_ref[...], we1[...], preferred_element_type=f32)
        e2_ref[...] = jnp.dot(emb_ref[...], we2[...], preferred_element_type=f32)


def _mid_body(agg_ref, den_ref, skip_ref, wq, wk, wv, ws, bias_ref,
              q_ref, kv_ref, skip2_ref):
    f32 = jnp.float32
    den = den_ref[0, :, 0:1] + den_ref[1, :, 0:1]
    agg = agg_ref[0] + agg_ref[1]
    h = jnp.maximum(agg / (den + 1e-16) + skip_ref[...], 0.0)
    q_ref[...] = jnp.dot(h, wq[...], preferred_element_type=f32) + bias_ref[0:1, :]
    kv_ref[:, :_H] = jnp.dot(h, wk[...], preferred_element_type=f32) + bias_ref[1:2, :]
    kv_ref[:, _H:] = jnp.dot(h, wv[...], preferred_element_type=f32) + bias_ref[2:3, :]
    skip2_ref[...] = jnp.dot(h, ws[...], preferred_element_type=f32) + bias_ref[3:4, :]


def _final_body(agg_ref, den_ref, skip_ref, wo, bias_ref, out_ref):
    den = den_ref[0, :, 0:1] + den_ref[1, :, 0:1]
    agg = agg_ref[0] + agg_ref[1]
    h = jnp.maximum(agg / (den + 1e-16) + skip_ref[...], 0.0)
    out_ref[...] = (jnp.dot(h, wo[...], preferred_element_type=jnp.float32)
                    + bias_ref[0:1, :2])


def _full_spec(shape):
    return pl.BlockSpec(shape, lambda i: tuple(0 for _ in shape))


def _proj1(x, emb, wq, wk, wv, ws, we1, we2, bias):
    n, d = x.shape
    blk = 1024
    grid = (n // blk,)
    row = lambda i: (i, 0)
    return pl.pallas_call(
        _proj1_body,
        grid=grid,
        in_specs=[
            pl.BlockSpec((blk, d), row),
            _full_spec(emb.shape),
            _full_spec(wq.shape), _full_spec(wk.shape),
            _full_spec(wv.shape), _full_spec(ws.shape),
            _full_spec(we1.shape), _full_spec(we2.shape),
            _full_spec(bias.shape),
        ],
        out_specs=[
            pl.BlockSpec((blk, _H), row),
            pl.BlockSpec((blk, 2 * _H), row),
            pl.BlockSpec((blk, _H), row),
            _full_spec((16, _H)),
            _full_spec((16, _H)),
        ],
        out_shape=[
            jax.ShapeDtypeStruct((n, _H), jnp.float32),
            jax.ShapeDtypeStruct((n, 2 * _H), jnp.float32),
            jax.ShapeDtypeStruct((n, _H), jnp.float32),
            jax.ShapeDtypeStruct((16, _H), jnp.float32),
            jax.ShapeDtypeStruct((16, _H), jnp.float32),
        ],
    )(x, emb, wq, wk, wv, ws, we1, we2, bias)


def _mid(aggp, denp, skip, wq, wk, wv, ws, bias):
    n = skip.shape[0]
    blk = 1024
    grid = (n // blk,)
    row = lambda i: (i, 0)
    row3 = lambda i: (0, i, 0)
    return pl.pallas_call(
        _mid_body,
        grid=grid,
        in_specs=[
            pl.BlockSpec((2, blk, _H), row3),
            pl.BlockSpec((2, blk, _LANES), row3),
            pl.BlockSpec((blk, _H), row),
            _full_spec(wq.shape), _full_spec(wk.shape),
            _full_spec(wv.shape), _full_spec(ws.shape),
            _full_spec(bias.shape),
        ],
        out_specs=[
            pl.BlockSpec((blk, _H), row),
            pl.BlockSpec((blk, 2 * _H), row),
            pl.BlockSpec((blk, _H), row),
        ],
        out_shape=[
            jax.ShapeDtypeStruct((n, _H), jnp.float32),
            jax.ShapeDtypeStruct((n, 2 * _H), jnp.float32),
            jax.ShapeDtypeStruct((n, _H), jnp.float32),
        ],
    )(aggp, denp, skip, wq, wk, wv, ws, bias)


def _final(aggp, denp, skip, wo, bias):
    n = skip.shape[0]
    blk = 1024
    grid = (n // blk,)
    row = lambda i: (i, 0)
    row3 = lambda i: (0, i, 0)
    return pl.pallas_call(
        _final_body,
        grid=grid,
        in_specs=[
            pl.BlockSpec((2, blk, _H), row3),
            pl.BlockSpec((2, blk, _LANES), row3),
            pl.BlockSpec((blk, _H), row),
            _full_spec(wo.shape),
            _full_spec(bias.shape),
        ],
        out_specs=pl.BlockSpec((blk, 2), row),
        out_shape=jax.ShapeDtypeStruct((n, 2), jnp.float32),
    )(aggp, denp, skip, wo, bias)


def _edge_phase(qT, kvT, eT, src, dst, attr):
    n_nodes = qT.shape[0]
    n_edges = src.shape[0]
    nchunk = n_edges // _C
    max_chunks_per_tile = -(-nchunk // _NTILES)
    rows_per_tile = n_nodes // 16
    mesh = plsc.VectorSubcoreMesh(core_axis_name="c", subcore_axis_name="s")
    inv_sqrt_h = np.float32(1.0 / np.sqrt(_H))
    cp = pltpu.CompilerParams()
    for fld, val in (("needs_layout_passes", False),
                     ("use_tc_tiling_on_sc", False)):
        if fld in pltpu.CompilerParams.__dataclass_fields__:
            cp = dataclasses.replace(cp, **{fld: val})

    @functools.partial(
        pl.kernel,
        mesh=mesh,
        compiler_params=cp,
        out_type=[
            jax.ShapeDtypeStruct((2, n_nodes, _H), jnp.float32),
            jax.ShapeDtypeStruct((2, n_nodes, _LANES), jnp.float32),
        ],
        scratch_types=[
            pltpu.VMEM((_C,), jnp.int32),
            pltpu.VMEM((_C,), jnp.int32),
            pltpu.VMEM((_C,), jnp.int32),
            pltpu.VMEM((16, _H), jnp.float32),
            pltpu.VMEM((_C, _H), jnp.float32),
            pltpu.VMEM((_C, 2 * _H), jnp.float32),
            pltpu.VMEM((_C, _H), jnp.float32),
            pltpu.VMEM((_C, _LANES), jnp.float32),
            pltpu.VMEM_SHARED((n_nodes, _H), jnp.float32),
            pltpu.VMEM_SHARED((n_nodes, _LANES), jnp.float32),
        ],
    )
    def k(q_hbm, kv_hbm, e_hbm, s_hbm, d_hbm, a_hbm, agg_out, den_out,
          sidx, didx, aidx, ebuf, qbuf, kvbuf, obuf, dbuf, agg_s, den_s):
        cid = lax.axis_index("c")
        sid = lax.axis_index("s")
        wid = sid * 2 + cid

        zero16 = jnp.zeros((_LANES,), jnp.float32)

        @pl.loop(0, _C)
        def _(r):
            dbuf[r, pl.ds(0, _LANES)] = zero16

            @pl.loop(0, _H, step=_LANES)
            def _(j):
                obuf[r, pl.ds(j, _LANES)] = zero16

        @pl.loop(0, rows_per_tile // _C)
        def _(t):
            r0 = sid * rows_per_tile + t * _C
            pltpu.sync_copy(obuf.at[pl.ds(0, _C)], agg_s.at[pl.ds(r0, _C)])
            pltpu.sync_copy(dbuf.at[pl.ds(0, _C)], den_s.at[pl.ds(r0, _C)])

        pltpu.sync_copy(e_hbm, ebuf)
        plsc.subcore_barrier()

        lanes = lax.iota(jnp.int32, _LANES)

        @pl.loop(0, max_chunks_per_tile)
        def _(i):
            c = wid + i * _NTILES

            @pl.when(c < nchunk)
            def _():
                base = c * _C
                pltpu.sync_copy(s_hbm.at[pl.ds(base, _C)], sidx)
                pltpu.sync_copy(d_hbm.at[pl.ds(base, _C)], didx)
                pltpu.sync_copy(a_hbm.at[pl.ds(base, _C)], aidx)
                pltpu.sync_copy(q_hbm.at[didx], qbuf)
                pltpu.sync_copy(kv_hbm.at[sidx], kvbuf)

                @pl.loop(0, _C // _LANES)
                def _(g):
                    rows = lanes + g * _LANES
                    attrv = aidx[pl.ds(g * _LANES, _LANES)]

                    def dot_body(j, acc):
                        jv = jnp.zeros((_LANES,), jnp.int32) + j
                        qv = plsc.load_gather(qbuf, [rows, jv])
                        kv_ = plsc.load_gather(kvbuf, [rows, jv])
                        ev = plsc.load_gather(ebuf, [attrv, jv])
                        return acc + qv * (kv_ + ev)

                    acc = lax.fori_loop(0, _H, dot_body,
                                        jnp.zeros((_LANES,), jnp.float32))
                    ex = jnp.exp(acc * inv_sqrt_h)
                    plsc.store_scatter(
                        dbuf, [rows, jnp.zeros((_LANES,), jnp.int32)], ex)

                    def v_body(j, _):
                        jv = jnp.zeros((_LANES,), jnp.int32) + j
                        vv = plsc.load_gather(kvbuf, [rows, jv + _H])
                        ev = plsc.load_gather(ebuf, [attrv, jv])
                        plsc.store_scatter(obuf, [rows, jv], ex * (vv + ev))
                        return 0

                    lax.fori_loop(0, _H, v_body, 0)

                pltpu.sync_copy(obuf, agg_s.at[didx], add=True)
                pltpu.sync_copy(dbuf, den_s.at[didx], add=True)

        plsc.subcore_barrier()
        r0 = sid * rows_per_tile
        pltpu.sync_copy(agg_s.at[pl.ds(r0, rows_per_tile)],
                        agg_out.at[cid, pl.ds(r0, rows_per_tile)])
        pltpu.sync_copy(den_s.at[pl.ds(r0, rows_per_tile)],
                        den_out.at[cid, pl.ds(r0, rows_per_tile)])

    return k(qT, kvT, eT, src, dst, attr)


def kernel(x, edge_index, edge_attr, emb, W1q, b1q, W1k, b1k, W1v, b1v, W1e,
           W1s, b1s, W2q, b2q, W2k, b2k, W2v, b2v, W2e, W2s, b2s, Wo, bo):
    src = edge_index[0]
    dst = edge_index[1]
    attr = edge_attr.astype(jnp.int32)
    n = x.shape[0]
    npad = -(-n // (16 * _C)) * (16 * _C)
    x = jnp.pad(x, ((0, npad - n), (0, 0)))

    bias1 = jnp.zeros((8, _H), jnp.float32)
    bias1 = bias1.at[0].set(b1q).at[1].set(b1k).at[2].set(b1v).at[3].set(b1s)
    bias2 = jnp.zeros((8, _H), jnp.float32)
    bias2 = bias2.at[0].set(b2q).at[1].set(b2k).at[2].set(b2v).at[3].set(b2s)
    biaso = jnp.zeros((8, _H), jnp.float32)
    biaso = biaso.at[0, :2].set(bo)

    qT1, kvT1, skip1, e1, e2 = _proj1(x, emb, W1q, W1k, W1v, W1s, W1e, W2e,
                                      bias1)
    aggp1, denp1 = _edge_phase(qT1, kvT1, e1, src, dst, attr)
    qT2, kvT2, skip2 = _mid(aggp1, denp1, skip1, W2q, W2k, W2v, W2s, bias2)
    aggp2, denp2 = _edge_phase(qT2, kvT2, e2, src, dst, attr)
    return _final(aggp2, denp2, skip2, Wo, biaso)[:n]


# speedup vs baseline: 3.1595x; 1.0155x over previous
import dataclasses
import functools

import jax
import jax.numpy as jnp
import numpy as np
from jax import lax
from jax.experimental import pallas as pl
from jax.experimental.pallas import tpu as pltpu
from jax.experimental.pallas import tpu_sc as plsc

_H = 64
_C = 128
_NTILES = 32
_LANES = 16


def _proj1_body(x_ref, emb_ref, wq, wk, wv, ws, we1, we2, bias_ref,
                q_ref, kv_ref, skip_ref, e1_ref, e2_ref):
    xb = x_ref[...]
    f32 = jnp.float32
    q_ref[...] = jnp.dot(xb, wq[...], preferred_element_type=f32) + bias_ref[0:1, :]
    kv_ref[:, :_H] = jnp.dot(xb, wk[...], preferred_element_type=f32) + bias_ref[1:2, :]
    kv_ref[:, _H:] = jnp.dot(xb, wv[...], preferred_element_type=f32) + bias_ref[2:3, :]
    skip_ref[...] = jnp.dot(xb, ws[...], preferred_element_type=f32) + bias_ref[3:4, :]

    @pl.when(pl.program_id(0) == 0)
    def _():
        e1_ref[...] = jnp.dot(emb_ref[...], we1[...], preferred_element_type=f32)
        e2_ref[...] = jnp.dot(emb_ref[...], we2[...], preferred_element_type=f32)


def _mid_body(agg_ref, den_ref, skip_ref, wq, wk, wv, ws, bias_ref,
              q_ref, kv_ref, skip2_ref):
    f32 = jnp.float32
    den = den_ref[0, :, 0:1] + den_ref[1, :, 0:1]
    agg = agg_ref[0] + agg_ref[1]
    h = jnp.maximum(agg / (den + 1e-16) + skip_ref[...], 0.0)
    q_ref[...] = jnp.dot(h, wq[...], preferred_element_type=f32) + bias_ref[0:1, :]
    kv_ref[:, :_H] = jnp.dot(h, wk[...], preferred_element_type=f32) + bias_ref[1:2, :]
    kv_ref[:, _H:] = jnp.dot(h, wv[...], preferred_element_type=f32) + bias_ref[2:3, :]
    skip2_ref[...] = jnp.dot(h, ws[...], preferred_element_type=f32) + bias_ref[3:4, :]


def _final_body(agg_ref, den_ref, skip_ref, wo, bias_ref, out_ref):
    den = den_ref[0, :, 0:1] + den_ref[1, :, 0:1]
    agg = agg_ref[0] + agg_ref[1]
    h = jnp.maximum(agg / (den + 1e-16) + skip_ref[...], 0.0)
    out_ref[...] = (jnp.dot(h, wo[...], preferred_element_type=jnp.float32)
                    + bias_ref[0:1, :2])


def _full_spec(shape):
    return pl.BlockSpec(shape, lambda i: tuple(0 for _ in shape))


def _proj1(x, emb, wq, wk, wv, ws, we1, we2, bias):
    n, d = x.shape
    blk = 1024
    grid = (n // blk,)
    row = lambda i: (i, 0)
    return pl.pallas_call(
        _proj1_body,
        grid=grid,
        in_specs=[
            pl.BlockSpec((blk, d), row),
            _full_spec(emb.shape),
            _full_spec(wq.shape), _full_spec(wk.shape),
            _full_spec(wv.shape), _full_spec(ws.shape),
            _full_spec(we1.shape), _full_spec(we2.shape),
            _full_spec(bias.shape),
        ],
        out_specs=[
            pl.BlockSpec((blk, _H), row),
            pl.BlockSpec((blk, 2 * _H), row),
            pl.BlockSpec((blk, _H), row),
            _full_spec((16, _H)),
            _full_spec((16, _H)),
        ],
        out_shape=[
            jax.ShapeDtypeStruct((n, _H), jnp.float32),
            jax.ShapeDtypeStruct((n, 2 * _H), jnp.float32),
            jax.ShapeDtypeStruct((n, _H), jnp.float32),
            jax.ShapeDtypeStruct((16, _H), jnp.float32),
            jax.ShapeDtypeStruct((16, _H), jnp.float32),
        ],
    )(x, emb, wq, wk, wv, ws, we1, we2, bias)


def _mid(aggp, denp, skip, wq, wk, wv, ws, bias):
    n = skip.shape[0]
    blk = 1024
    grid = (n // blk,)
    row = lambda i: (i, 0)
    row3 = lambda i: (0, i, 0)
    return pl.pallas_call(
        _mid_body,
        grid=grid,
        in_specs=[
            pl.BlockSpec((2, blk, _H), row3),
            pl.BlockSpec((2, blk, _LANES), row3),
            pl.BlockSpec((blk, _H), row),
            _full_spec(wq.shape), _full_spec(wk.shape),
            _full_spec(wv.shape), _full_spec(ws.shape),
            _full_spec(bias.shape),
        ],
        out_specs=[
            pl.BlockSpec((blk, _H), row),
            pl.BlockSpec((blk, 2 * _H), row),
            pl.BlockSpec((blk, _H), row),
        ],
        out_shape=[
            jax.ShapeDtypeStruct((n, _H), jnp.float32),
            jax.ShapeDtypeStruct((n, 2 * _H), jnp.float32),
            jax.ShapeDtypeStruct((n, _H), jnp.float32),
        ],
    )(aggp, denp, skip, wq, wk, wv, ws, bias)


def _final(aggp, denp, skip, wo, bias):
    n = skip.shape[0]
    blk = 1024
    grid = (n // blk,)
    row = lambda i: (i, 0)
    row3 = lambda i: (0, i, 0)
    return pl.pallas_call(
        _final_body,
        grid=grid,
        in_specs=[
            pl.BlockSpec((2, blk, _H), row3),
            pl.BlockSpec((2, blk, _LANES), row3),
            pl.BlockSpec((blk, _H), row),
            _full_spec(wo.shape),
            _full_spec(bias.shape),
        ],
        out_specs=pl.BlockSpec((blk, 2), row),
        out_shape=jax.ShapeDtypeStruct((n, 2), jnp.float32),
    )(aggp, denp, skip, wo, bias)


def _edge_phase(qT, kvT, eT, src, dst, attr):
    n_nodes = qT.shape[0]
    n_edges = src.shape[0]
    nchunk = n_edges // _C
    chunks_per_tile = nchunk // _NTILES
    rows_per_tile = n_nodes // 16
    mesh = plsc.VectorSubcoreMesh(core_axis_name="c", subcore_axis_name="s")
    inv_sqrt_h = np.float32(1.0 / np.sqrt(_H))
    cp = pltpu.CompilerParams()
    for fld, val in (("needs_layout_passes", False),
                     ("use_tc_tiling_on_sc", False)):
        if fld in pltpu.CompilerParams.__dataclass_fields__:
            cp = dataclasses.replace(cp, **{fld: val})

    @functools.partial(
        pl.kernel,
        mesh=mesh,
        compiler_params=cp,
        out_type=[
            jax.ShapeDtypeStruct((2, n_nodes, _H), jnp.float32),
            jax.ShapeDtypeStruct((2, n_nodes, _LANES), jnp.float32),
        ],
        scratch_types=[
            pltpu.VMEM((_C,), jnp.int32), pltpu.VMEM((_C,), jnp.int32),
            pltpu.VMEM((_C,), jnp.int32), pltpu.VMEM((_C,), jnp.int32),
            pltpu.VMEM((_C,), jnp.int32), pltpu.VMEM((_C,), jnp.int32),
            pltpu.VMEM((_C,), jnp.int32), pltpu.VMEM((_C,), jnp.int32),
            pltpu.VMEM((16, _H), jnp.float32),
            pltpu.VMEM((_C, _H), jnp.float32),
            pltpu.VMEM((_C, _H), jnp.float32),
            pltpu.VMEM((_C, 2 * _H), jnp.float32),
            pltpu.VMEM((_C, 2 * _H), jnp.float32),
            pltpu.VMEM((_C, _H), jnp.float32),
            pltpu.VMEM((_C, _H), jnp.float32),
            pltpu.VMEM((_C, _LANES), jnp.float32),
            pltpu.VMEM((_C, _LANES), jnp.float32),
            pltpu.VMEM_SHARED((n_nodes, _H), jnp.float32),
            pltpu.VMEM_SHARED((n_nodes, _LANES), jnp.float32),
            pltpu.SemaphoreType.DMA, pltpu.SemaphoreType.DMA,
            pltpu.SemaphoreType.DMA, pltpu.SemaphoreType.DMA,
            pltpu.SemaphoreType.DMA, pltpu.SemaphoreType.DMA,
        ],
    )
    def k(q_hbm, kv_hbm, e_hbm, s_hbm, d_hbm, a_hbm, agg_out, den_out,
          sidx0, sidx1, didx0, didx1, aidx0, aidx1, scat0, scat1,
          ebuf, qbuf0, qbuf1, kvbuf0, kvbuf1, obuf0, obuf1, dbuf0, dbuf1,
          agg_s, den_s,
          semi0, semi1, semg0, semg1, sems0, sems1):
        cid = lax.axis_index("c")
        sid = lax.axis_index("s")
        wid = sid * 2 + cid

        sidx = (sidx0, sidx1)
        didx = (didx0, didx1)
        aidx = (aidx0, aidx1)
        scat = (scat0, scat1)
        qbuf = (qbuf0, qbuf1)
        kvbuf = (kvbuf0, kvbuf1)
        obuf = (obuf0, obuf1)
        dbuf = (dbuf0, dbuf1)
        semi = (semi0, semi1)
        semg = (semg0, semg1)
        sems = (sems0, sems1)

        zero16 = jnp.zeros((_LANES,), jnp.float32)

        @pl.loop(0, _C)
        def _(r):
            dbuf0[r, pl.ds(0, _LANES)] = zero16
            dbuf1[r, pl.ds(0, _LANES)] = zero16

            @pl.loop(0, _H, step=_LANES)
            def _(j):
                obuf0[r, pl.ds(j, _LANES)] = zero16

        @pl.loop(0, rows_per_tile // _C)
        def _(t):
            r0 = sid * rows_per_tile + t * _C
            pltpu.sync_copy(obuf0.at[pl.ds(0, _C)], agg_s.at[pl.ds(r0, _C)])
            pltpu.sync_copy(dbuf0.at[pl.ds(0, _C)], den_s.at[pl.ds(r0, _C)])

        pltpu.sync_copy(e_hbm, ebuf)
        plsc.subcore_barrier()

        lanes = lax.iota(jnp.int32, _LANES)

        def idx_copies(t, b):
            base = (wid + t * _NTILES) * _C
            return (
                pltpu.make_async_copy(s_hbm.at[pl.ds(base, _C)], sidx[b], semi[b]),
                pltpu.make_async_copy(d_hbm.at[pl.ds(base, _C)], didx[b], semi[b]),
                pltpu.make_async_copy(a_hbm.at[pl.ds(base, _C)], aidx[b], semi[b]),
            )

        def gather_copies(b):
            return (
                pltpu.make_async_copy(q_hbm.at[didx[b]], qbuf[b], semg[b]),
                pltpu.make_async_copy(kv_hbm.at[sidx[b]], kvbuf[b], semg[b]),
            )

        def scatter_copies(b):
            return (
                pltpu.make_async_copy(obuf[b], agg_s.at[scat[b]], sems[b]),
                pltpu.make_async_copy(dbuf[b], den_s.at[scat[b]], sems[b]),
            )

        def start(copies, add=False):
            for c in copies:
                c.start(add=add)

        def wait(copies):
            for c in copies:
                c.wait()

        def compute(b):
            @pl.loop(0, _C // _LANES)
            def _(g):
                rows = lanes + g * _LANES
                attrv = aidx[b][pl.ds(g * _LANES, _LANES)]

                def dot_body(j, acc):
                    jv = jnp.zeros((_LANES,), jnp.int32) + j
                    qv = plsc.load_gather(qbuf[b], [rows, jv])
                    kv_ = plsc.load_gather(kvbuf[b], [rows, jv])
                    ev = plsc.load_gather(ebuf, [attrv, jv])
                    return acc + qv * (kv_ + ev)

                acc = lax.fori_loop(0, _H, dot_body,
                                    jnp.zeros((_LANES,), jnp.float32))
                ex = jnp.exp(acc * inv_sqrt_h)
                plsc.store_scatter(
                    dbuf[b], [rows, jnp.zeros((_LANES,), jnp.int32)], ex)

                def v_body(j, _):
                    jv = jnp.zeros((_LANES,), jnp.int32) + j
                    vv = plsc.load_gather(kvbuf[b], [rows, jv + _H])
                    ev = plsc.load_gather(ebuf, [attrv, jv])
                    plsc.store_scatter(obuf[b], [rows, jv], ex * (vv + ev))
                    return 0

                lax.fori_loop(0, _H, v_body, 0)

        def stage(t, b):
            wait(gather_copies(b))
            @pl.when(t + 1 < chunks_per_tile)
            def _():
                wait(idx_copies(t + 1, 1 - b))
                start(gather_copies(1 - b))
            @pl.when(t >= 2)
            def _():
                wait(scatter_copies(b))
            compute(b)

            @pl.loop(0, _C, step=_LANES)
            def _(r):
                scat[b][pl.ds(r, _LANES)] = didx[b][pl.ds(r, _LANES)]

            start(scatter_copies(b), add=True)

            @pl.when(t + 2 < chunks_per_tile)
            def _():
                start(idx_copies(t + 2, b))

        start(idx_copies(0, 0))
        start(idx_copies(1, 1))
        wait(idx_copies(0, 0))
        start(gather_copies(0))

        @pl.loop(0, chunks_per_tile, step=2)
        def _(i):
            stage(i, 0)
            stage(i + 1, 1)

        wait(scatter_copies(0))
        wait(scatter_copies(1))

        plsc.subcore_barrier()
        r0 = sid * rows_per_tile
        pltpu.sync_copy(agg_s.at[pl.ds(r0, rows_per_tile)],
                        agg_out.at[cid, pl.ds(r0, rows_per_tile)])
        pltpu.sync_copy(den_s.at[pl.ds(r0, rows_per_tile)],
                        den_out.at[cid, pl.ds(r0, rows_per_tile)])

    return k(qT, kvT, eT, src, dst, attr)


def kernel(x, edge_index, edge_attr, emb, W1q, b1q, W1k, b1k, W1v, b1v, W1e,
           W1s, b1s, W2q, b2q, W2k, b2k, W2v, b2v, W2e, W2s, b2s, Wo, bo):
    src = edge_index[0]
    dst = edge_index[1]
    attr = edge_attr.astype(jnp.int32)
    n = x.shape[0]
    npad = -(-n // (16 * _C)) * (16 * _C)
    x = jnp.pad(x, ((0, npad - n), (0, 0)))

    e = src.shape[0]
    epad = -(-e // (2 * _NTILES * _C)) * (2 * _NTILES * _C)
    src = jnp.pad(src, (0, epad - e))
    dst = jnp.pad(dst, (0, epad - e), constant_values=npad - 1)
    attr = jnp.pad(attr, (0, epad - e))

    bias1 = jnp.zeros((8, _H), jnp.float32)
    bias1 = bias1.at[0].set(b1q).at[1].set(b1k).at[2].set(b1v).at[3].set(b1s)
    bias2 = jnp.zeros((8, _H), jnp.float32)
    bias2 = bias2.at[0].set(b2q).at[1].set(b2k).at[2].set(b2v).at[3].set(b2s)
    biaso = jnp.zeros((8, _H), jnp.float32)
    biaso = biaso.at[0, :2].set(bo)

    qT1, kvT1, skip1, e1, e2 = _proj1(x, emb, W1q, W1k, W1v, W1s, W1e, W2e,
                                      bias1)
    aggp1, denp1 = _edge_phase(qT1, kvT1, e1, src, dst, attr)
    qT2, kvT2, skip2 = _mid(aggp1, denp1, skip1, W2q, W2k, W2v, W2s, bias2)
    aggp2, denp2 = _edge_phase(qT2, kvT2, e2, src, dst, attr)
    return _final(aggp2, denp2, skip2, Wo, biaso)[:n]


# speedup vs baseline: 4.8798x; 1.5445x over previous
import dataclasses
import functools

import jax
import jax.numpy as jnp
import numpy as np
from jax import lax
from jax.experimental import pallas as pl
from jax.experimental.pallas import tpu as pltpu
from jax.experimental.pallas import tpu_sc as plsc

_H = 64
_C = 128
_NTILES = 32
_LANES = 16


def _proj1_body(x_ref, emb_ref, wq, wk, wv, ws, we1, we2, bias_ref,
                qx_ref, kv_ref, skip_ref, e1_ref, e2_ref):
    xb = x_ref[...]
    f32 = jnp.float32
    e1 = jnp.dot(emb_ref[...], we1[...], preferred_element_type=f32)
    q = jnp.dot(xb, wq[...], preferred_element_type=f32) + bias_ref[0:1, :]
    qx_ref[:, :_H] = q
    qx_ref[:, _H:] = jnp.dot(q, e1.T, preferred_element_type=f32,
                             precision=jax.lax.Precision.HIGHEST)
    kv_ref[:, :_H] = jnp.dot(xb, wk[...], preferred_element_type=f32) + bias_ref[1:2, :]
    kv_ref[:, _H:] = jnp.dot(xb, wv[...], preferred_element_type=f32) + bias_ref[2:3, :]
    skip_ref[...] = jnp.dot(xb, ws[...], preferred_element_type=f32) + bias_ref[3:4, :]

    @pl.when(pl.program_id(0) == 0)
    def _():
        e1_ref[...] = e1
        e2_ref[...] = jnp.dot(emb_ref[...], we2[...], preferred_element_type=f32)


def _combine(agg_ref, s_ref, skip_ref, e_ref):
    s = s_ref[0] + s_ref[1]
    den = jnp.sum(s, axis=-1, keepdims=True)
    agg = (agg_ref[0] + agg_ref[1]
           + jnp.dot(s, e_ref[...], preferred_element_type=jnp.float32,
                     precision=jax.lax.Precision.HIGHEST))
    return jnp.maximum(agg / (den + 1e-16) + skip_ref[...], 0.0)


def _mid_body(agg_ref, s_ref, skip_ref, e1_ref, e2_ref, wq, wk, wv, ws,
              bias_ref, qx_ref, kv_ref, skip2_ref):
    f32 = jnp.float32
    h = _combine(agg_ref, s_ref, skip_ref, e1_ref)
    q = jnp.dot(h, wq[...], preferred_element_type=f32) + bias_ref[0:1, :]
    qx_ref[:, :_H] = q
    qx_ref[:, _H:] = jnp.dot(q, e2_ref[...].T, preferred_element_type=f32,
                             precision=jax.lax.Precision.HIGHEST)
    kv_ref[:, :_H] = jnp.dot(h, wk[...], preferred_element_type=f32) + bias_ref[1:2, :]
    kv_ref[:, _H:] = jnp.dot(h, wv[...], preferred_element_type=f32) + bias_ref[2:3, :]
    skip2_ref[...] = jnp.dot(h, ws[...], preferred_element_type=f32) + bias_ref[3:4, :]


def _final_body(agg_ref, s_ref, skip_ref, e2_ref, wo, bias_ref, out_ref):
    h = _combine(agg_ref, s_ref, skip_ref, e2_ref)
    out_ref[...] = (jnp.dot(h, wo[...], preferred_element_type=jnp.float32)
                    + bias_ref[0:1, :2])


def _full_spec(shape):
    return pl.BlockSpec(shape, lambda i: tuple(0 for _ in shape))


def _proj1(x, emb, wq, wk, wv, ws, we1, we2, bias):
    n, d = x.shape
    blk = 1024
    grid = (n // blk,)
    row = lambda i: (i, 0)
    return pl.pallas_call(
        _proj1_body,
        grid=grid,
        in_specs=[
            pl.BlockSpec((blk, d), row),
            _full_spec(emb.shape),
            _full_spec(wq.shape), _full_spec(wk.shape),
            _full_spec(wv.shape), _full_spec(ws.shape),
            _full_spec(we1.shape), _full_spec(we2.shape),
            _full_spec(bias.shape),
        ],
        out_specs=[
            pl.BlockSpec((blk, _H + _LANES), row),
            pl.BlockSpec((blk, 2 * _H), row),
            pl.BlockSpec((blk, _H), row),
            _full_spec((16, _H)),
            _full_spec((16, _H)),
        ],
        out_shape=[
            jax.ShapeDtypeStruct((n, _H + _LANES), jnp.float32),
            jax.ShapeDtypeStruct((n, 2 * _H), jnp.float32),
            jax.ShapeDtypeStruct((n, _H), jnp.float32),
            jax.ShapeDtypeStruct((16, _H), jnp.float32),
            jax.ShapeDtypeStruct((16, _H), jnp.float32),
        ],
    )(x, emb, wq, wk, wv, ws, we1, we2, bias)


def _mid(aggp, sp, skip, e1, e2, wq, wk, wv, ws, bias):
    n = skip.shape[0]
    blk = 1024
    grid = (n // blk,)
    row = lambda i: (i, 0)
    row3 = lambda i: (0, i, 0)
    return pl.pallas_call(
        _mid_body,
        grid=grid,
        in_specs=[
            pl.BlockSpec((2, blk, _H), row3),
            pl.BlockSpec((2, blk, _LANES), row3),
            pl.BlockSpec((blk, _H), row),
            _full_spec(e1.shape), _full_spec(e2.shape),
            _full_spec(wq.shape), _full_spec(wk.shape),
            _full_spec(wv.shape), _full_spec(ws.shape),
            _full_spec(bias.shape),
        ],
        out_specs=[
            pl.BlockSpec((blk, _H + _LANES), row),
            pl.BlockSpec((blk, 2 * _H), row),
            pl.BlockSpec((blk, _H), row),
        ],
        out_shape=[
            jax.ShapeDtypeStruct((n, _H + _LANES), jnp.float32),
            jax.ShapeDtypeStruct((n, 2 * _H), jnp.float32),
            jax.ShapeDtypeStruct((n, _H), jnp.float32),
        ],
    )(aggp, sp, skip, e1, e2, wq, wk, wv, ws, bias)


def _final(aggp, sp, skip, e2, wo, bias):
    n = skip.shape[0]
    blk = 1024
    grid = (n // blk,)
    row = lambda i: (i, 0)
    row3 = lambda i: (0, i, 0)
    return pl.pallas_call(
        _final_body,
        grid=grid,
        in_specs=[
            pl.BlockSpec((2, blk, _H), row3),
            pl.BlockSpec((2, blk, _LANES), row3),
            pl.BlockSpec((blk, _H), row),
            _full_spec(e2.shape),
            _full_spec(wo.shape),
            _full_spec(bias.shape),
        ],
        out_specs=pl.BlockSpec((blk, 2), row),
        out_shape=jax.ShapeDtypeStruct((n, 2), jnp.float32),
    )(aggp, sp, skip, e2, wo, bias)


def _edge_phase(qT, kvT, src, dst, attr):
    n_nodes = qT.shape[0]
    n_edges = src.shape[0]
    nchunk = n_edges // _C
    chunks_per_tile = nchunk // _NTILES
    rows_per_tile = n_nodes // 16
    mesh = plsc.VectorSubcoreMesh(core_axis_name="c", subcore_axis_name="s")
    inv_sqrt_h = np.float32(1.0 / np.sqrt(_H))
    cp = pltpu.CompilerParams()
    for fld, val in (("needs_layout_passes", False),
                     ("use_tc_tiling_on_sc", False)):
        if fld in pltpu.CompilerParams.__dataclass_fields__:
            cp = dataclasses.replace(cp, **{fld: val})

    @functools.partial(
        pl.kernel,
        mesh=mesh,
        compiler_params=cp,
        out_type=[
            jax.ShapeDtypeStruct((2, n_nodes, _H), jnp.float32),
            jax.ShapeDtypeStruct((2, n_nodes, _LANES), jnp.float32),
        ],
        scratch_types=[
            pltpu.VMEM((_C,), jnp.int32), pltpu.VMEM((_C,), jnp.int32),
            pltpu.VMEM((_C,), jnp.int32), pltpu.VMEM((_C,), jnp.int32),
            pltpu.VMEM((_C,), jnp.int32), pltpu.VMEM((_C,), jnp.int32),
            pltpu.VMEM((_C,), jnp.int32), pltpu.VMEM((_C,), jnp.int32),
            pltpu.VMEM((_C, _H + _LANES), jnp.float32),
            pltpu.VMEM((_C, _H + _LANES), jnp.float32),
            pltpu.VMEM((_C, 2 * _H), jnp.float32),
            pltpu.VMEM((_C, 2 * _H), jnp.float32),
            pltpu.VMEM((_C, _H), jnp.float32),
            pltpu.VMEM((_C, _H), jnp.float32),
            pltpu.VMEM((_C, _LANES), jnp.float32),
            pltpu.VMEM((_C, _LANES), jnp.float32),
            pltpu.VMEM_SHARED((n_nodes, _H), jnp.float32),
            pltpu.VMEM_SHARED((n_nodes, _LANES), jnp.float32),
            pltpu.SemaphoreType.DMA, pltpu.SemaphoreType.DMA,
            pltpu.SemaphoreType.DMA, pltpu.SemaphoreType.DMA,
            pltpu.SemaphoreType.DMA, pltpu.SemaphoreType.DMA,
        ],
    )
    def k(q_hbm, kv_hbm, s_hbm, d_hbm, a_hbm, agg_out, den_out,
          sidx0, sidx1, didx0, didx1, aidx0, aidx1, scat0, scat1,
          qbuf0, qbuf1, kvbuf0, kvbuf1, obuf0, obuf1, dbuf0, dbuf1,
          agg_s, den_s,
          semi0, semi1, semg0, semg1, sems0, sems1):
        cid = lax.axis_index("c")
        sid = lax.axis_index("s")
        wid = sid * 2 + cid

        sidx = (sidx0, sidx1)
        didx = (didx0, didx1)
        aidx = (aidx0, aidx1)
        scat = (scat0, scat1)
        qbuf = (qbuf0, qbuf1)
        kvbuf = (kvbuf0, kvbuf1)
        obuf = (obuf0, obuf1)
        dbuf = (dbuf0, dbuf1)
        semi = (semi0, semi1)
        semg = (semg0, semg1)
        sems = (sems0, sems1)

        zero16 = jnp.zeros((_LANES,), jnp.float32)

        @pl.loop(0, _C)
        def _(r):
            dbuf0[r, pl.ds(0, _LANES)] = zero16
            dbuf1[r, pl.ds(0, _LANES)] = zero16

            @pl.loop(0, _H, step=_LANES)
            def _(j):
                obuf0[r, pl.ds(j, _LANES)] = zero16

        @pl.loop(0, rows_per_tile // _C)
        def _(t):
            r0 = sid * rows_per_tile + t * _C
            pltpu.sync_copy(obuf0.at[pl.ds(0, _C)], agg_s.at[pl.ds(r0, _C)])
            pltpu.sync_copy(dbuf0.at[pl.ds(0, _C)], den_s.at[pl.ds(r0, _C)])

        plsc.subcore_barrier()

        lanes = lax.iota(jnp.int32, _LANES)

        def idx_copies(t, b):
            base = (wid + t * _NTILES) * _C
            return (
                pltpu.make_async_copy(s_hbm.at[pl.ds(base, _C)], sidx[b], semi[b]),
                pltpu.make_async_copy(d_hbm.at[pl.ds(base, _C)], didx[b], semi[b]),
                pltpu.make_async_copy(a_hbm.at[pl.ds(base, _C)], aidx[b], semi[b]),
            )

        def gather_copies(b):
            return (
                pltpu.make_async_copy(q_hbm.at[didx[b]], qbuf[b], semg[b]),
                pltpu.make_async_copy(kv_hbm.at[sidx[b]], kvbuf[b], semg[b]),
            )

        def scatter_copies(b):
            return (
                pltpu.make_async_copy(obuf[b], agg_s.at[scat[b]], sems[b]),
                pltpu.make_async_copy(dbuf[b], den_s.at[scat[b]], sems[b]),
            )

        def start(copies, add=False):
            for c in copies:
                c.start(add=add)

        def wait(copies):
            for c in copies:
                c.wait()

        def compute(b):
            @pl.loop(0, _C, unroll=8)
            def _(r):
                dbuf[b][r, pl.ds(0, _LANES)] = zero16

            @pl.loop(0, _C // _LANES)
            def _(g):
                rows = lanes + g * _LANES
                attrv = aidx[b][pl.ds(g * _LANES, _LANES)]

                def dot_body(j, acc):
                    jv = jnp.zeros((_LANES,), jnp.int32) + j
                    qv = plsc.load_gather(qbuf[b], [rows, jv])
                    kv_ = plsc.load_gather(kvbuf[b], [rows, jv])
                    return acc + qv * kv_

                acc = lax.fori_loop(0, _H, dot_body,
                                    jnp.zeros((_LANES,), jnp.float32),
                                    unroll=8)
                qe = plsc.load_gather(qbuf[b], [rows, attrv + _H])
                ex = jnp.exp((acc + qe) * inv_sqrt_h)
                plsc.store_scatter(dbuf[b], [rows, attrv], ex)

                def v_body(j, _):
                    jv = jnp.zeros((_LANES,), jnp.int32) + j
                    vv = plsc.load_gather(kvbuf[b], [rows, jv + _H])
                    plsc.store_scatter(obuf[b], [rows, jv], ex * vv)
                    return 0

                lax.fori_loop(0, _H, v_body, 0, unroll=8)

        def stage(t, b):
            wait(gather_copies(b))
            @pl.when(t + 1 < chunks_per_tile)
            def _():
                wait(idx_copies(t + 1, 1 - b))
                start(gather_copies(1 - b))
            @pl.when(t >= 2)
            def _():
                wait(scatter_copies(b))
            compute(b)

            @pl.loop(0, _C, step=_LANES)
            def _(r):
                scat[b][pl.ds(r, _LANES)] = didx[b][pl.ds(r, _LANES)]

            start(scatter_copies(b), add=True)

            @pl.when(t + 2 < chunks_per_tile)
            def _():
                start(idx_copies(t + 2, b))

        start(idx_copies(0, 0))
        start(idx_copies(1, 1))
        wait(idx_copies(0, 0))
        start(gather_copies(0))

        @pl.loop(0, chunks_per_tile, step=2)
        def _(i):
            stage(i, 0)
            stage(i + 1, 1)

        wait(scatter_copies(0))
        wait(scatter_copies(1))

        plsc.subcore_barrier()
        r0 = sid * rows_per_tile
        pltpu.sync_copy(agg_s.at[pl.ds(r0, rows_per_tile)],
                        agg_out.at[cid, pl.ds(r0, rows_per_tile)])
        pltpu.sync_copy(den_s.at[pl.ds(r0, rows_per_tile)],
                        den_out.at[cid, pl.ds(r0, rows_per_tile)])

    return k(qT, kvT, src, dst, attr)


def kernel(x, edge_index, edge_attr, emb, W1q, b1q, W1k, b1k, W1v, b1v, W1e,
           W1s, b1s, W2q, b2q, W2k, b2k, W2v, b2v, W2e, W2s, b2s, Wo, bo):
    src = edge_index[0]
    dst = edge_index[1]
    attr = edge_attr.astype(jnp.int32)
    n = x.shape[0]
    npad = -(-n // (16 * _C)) * (16 * _C)
    x = jnp.pad(x, ((0, npad - n), (0, 0)))

    e = src.shape[0]
    epad = -(-e // (2 * _NTILES * _C)) * (2 * _NTILES * _C)
    src = jnp.pad(src, (0, epad - e))
    dst = jnp.pad(dst, (0, epad - e), constant_values=npad - 1)
    attr = jnp.pad(attr, (0, epad - e))

    bias1 = jnp.zeros((8, _H), jnp.float32)
    bias1 = bias1.at[0].set(b1q).at[1].set(b1k).at[2].set(b1v).at[3].set(b1s)
    bias2 = jnp.zeros((8, _H), jnp.float32)
    bias2 = bias2.at[0].set(b2q).at[1].set(b2k).at[2].set(b2v).at[3].set(b2s)
    biaso = jnp.zeros((8, _H), jnp.float32)
    biaso = biaso.at[0, :2].set(bo)

    qT1, kvT1, skip1, e1, e2 = _proj1(x, emb, W1q, W1k, W1v, W1s, W1e, W2e,
                                      bias1)
    aggp1, sp1 = _edge_phase(qT1, kvT1, src, dst, attr)
    qT2, kvT2, skip2 = _mid(aggp1, sp1, skip1, e1, e2, W2q, W2k, W2v, W2s,
                            bias2)
    aggp2, sp2 = _edge_phase(qT2, kvT2, src, dst, attr)
    return _final(aggp2, sp2, skip2, e2, Wo, biaso)[:n]


# speedup vs baseline: 5.8121x; 1.1910x over previous
import dataclasses
import functools

import jax
import jax.numpy as jnp
import numpy as np
from jax import lax
from jax.experimental import pallas as pl
from jax.experimental.pallas import tpu as pltpu
from jax.experimental.pallas import tpu_sc as plsc

_H = 64
_C = 128
_NTILES = 32
_LANES = 16


def _proj1_body(x_ref, emb_ref, wq, wk, wv, ws, we1, we2, bias_ref,
                qx_ref, kv_ref, skip_ref, e1_ref, e2_ref):
    xb = x_ref[...]
    f32 = jnp.float32
    e1 = jnp.dot(emb_ref[...], we1[...], preferred_element_type=f32)
    q = jnp.dot(xb, wq[...], preferred_element_type=f32) + bias_ref[0:1, :]
    qx_ref[:, :_H] = q
    qx_ref[:, _H:] = jnp.dot(q, e1.T, preferred_element_type=f32,
                             precision=jax.lax.Precision.HIGHEST)
    kv_ref[:, :_H] = jnp.dot(xb, wk[...], preferred_element_type=f32) + bias_ref[1:2, :]
    kv_ref[:, _H:] = jnp.dot(xb, wv[...], preferred_element_type=f32) + bias_ref[2:3, :]
    skip_ref[...] = jnp.dot(xb, ws[...], preferred_element_type=f32) + bias_ref[3:4, :]

    @pl.when(pl.program_id(0) == 0)
    def _():
        e1_ref[...] = e1
        e2_ref[...] = jnp.dot(emb_ref[...], we2[...], preferred_element_type=f32)


def _combine(acc_ref, skip_ref, e_ref):
    acc = acc_ref[0] + acc_ref[1]
    s = acc[:, _H:]
    den = jnp.sum(s, axis=-1, keepdims=True)
    agg = (acc[:, :_H]
           + jnp.dot(s, e_ref[...], preferred_element_type=jnp.float32,
                     precision=jax.lax.Precision.HIGHEST))
    return jnp.maximum(agg / (den + 1e-16) + skip_ref[...], 0.0)


def _mid_body(acc_ref, skip_ref, e1_ref, e2_ref, wq, wk, wv, ws,
              bias_ref, qx_ref, kv_ref, skip2_ref):
    f32 = jnp.float32
    h = _combine(acc_ref, skip_ref, e1_ref)
    q = jnp.dot(h, wq[...], preferred_element_type=f32) + bias_ref[0:1, :]
    qx_ref[:, :_H] = q
    qx_ref[:, _H:] = jnp.dot(q, e2_ref[...].T, preferred_element_type=f32,
                             precision=jax.lax.Precision.HIGHEST)
    kv_ref[:, :_H] = jnp.dot(h, wk[...], preferred_element_type=f32) + bias_ref[1:2, :]
    kv_ref[:, _H:] = jnp.dot(h, wv[...], preferred_element_type=f32) + bias_ref[2:3, :]
    skip2_ref[...] = jnp.dot(h, ws[...], preferred_element_type=f32) + bias_ref[3:4, :]


def _final_body(acc_ref, skip_ref, e2_ref, wo, bias_ref, out_ref):
    h = _combine(acc_ref, skip_ref, e2_ref)
    out_ref[...] = (jnp.dot(h, wo[...], preferred_element_type=jnp.float32)
                    + bias_ref[0:1, :2])


def _full_spec(shape):
    return pl.BlockSpec(shape, lambda i: tuple(0 for _ in shape))


def _proj1(x, emb, wq, wk, wv, ws, we1, we2, bias):
    n, d = x.shape
    blk = 1024
    grid = (n // blk,)
    row = lambda i: (i, 0)
    return pl.pallas_call(
        _proj1_body,
        grid=grid,
        in_specs=[
            pl.BlockSpec((blk, d), row),
            _full_spec(emb.shape),
            _full_spec(wq.shape), _full_spec(wk.shape),
            _full_spec(wv.shape), _full_spec(ws.shape),
            _full_spec(we1.shape), _full_spec(we2.shape),
            _full_spec(bias.shape),
        ],
        out_specs=[
            pl.BlockSpec((blk, _H + _LANES), row),
            pl.BlockSpec((blk, 2 * _H), row),
            pl.BlockSpec((blk, _H), row),
            _full_spec((16, _H)),
            _full_spec((16, _H)),
        ],
        out_shape=[
            jax.ShapeDtypeStruct((n, _H + _LANES), jnp.float32),
            jax.ShapeDtypeStruct((n, 2 * _H), jnp.float32),
            jax.ShapeDtypeStruct((n, _H), jnp.float32),
            jax.ShapeDtypeStruct((16, _H), jnp.float32),
            jax.ShapeDtypeStruct((16, _H), jnp.float32),
        ],
    )(x, emb, wq, wk, wv, ws, we1, we2, bias)


def _mid(accp, skip, e1, e2, wq, wk, wv, ws, bias):
    n = skip.shape[0]
    blk = 1024
    grid = (n // blk,)
    row = lambda i: (i, 0)
    row3 = lambda i: (0, i, 0)
    return pl.pallas_call(
        _mid_body,
        grid=grid,
        in_specs=[
            pl.BlockSpec((2, blk, _H + _LANES), row3),
            pl.BlockSpec((blk, _H), row),
            _full_spec(e1.shape), _full_spec(e2.shape),
            _full_spec(wq.shape), _full_spec(wk.shape),
            _full_spec(wv.shape), _full_spec(ws.shape),
            _full_spec(bias.shape),
        ],
        out_specs=[
            pl.BlockSpec((blk, _H + _LANES), row),
            pl.BlockSpec((blk, 2 * _H), row),
            pl.BlockSpec((blk, _H), row),
        ],
        out_shape=[
            jax.ShapeDtypeStruct((n, _H + _LANES), jnp.float32),
            jax.ShapeDtypeStruct((n, 2 * _H), jnp.float32),
            jax.ShapeDtypeStruct((n, _H), jnp.float32),
        ],
    )(accp, skip, e1, e2, wq, wk, wv, ws, bias)


def _final(accp, skip, e2, wo, bias):
    n = skip.shape[0]
    blk = 1024
    grid = (n // blk,)
    row = lambda i: (i, 0)
    row3 = lambda i: (0, i, 0)
    return pl.pallas_call(
        _final_body,
        grid=grid,
        in_specs=[
            pl.BlockSpec((2, blk, _H + _LANES), row3),
            pl.BlockSpec((blk, _H), row),
            _full_spec(e2.shape),
            _full_spec(wo.shape),
            _full_spec(bias.shape),
        ],
        out_specs=pl.BlockSpec((blk, 2), row),
        out_shape=jax.ShapeDtypeStruct((n, 2), jnp.float32),
    )(accp, skip, e2, wo, bias)


def _edge_phase(qT, kvT, src, dst, attr):
    n_nodes = qT.shape[0]
    n_edges = src.shape[0]
    nchunk = n_edges // _C
    chunks_per_tile = nchunk // _NTILES
    rows_per_tile = n_nodes // 16
    mesh = plsc.VectorSubcoreMesh(core_axis_name="c", subcore_axis_name="s")
    inv_sqrt_h = np.float32(1.0 / np.sqrt(_H))
    cp = pltpu.CompilerParams()
    for fld, val in (("needs_layout_passes", False),
                     ("use_tc_tiling_on_sc", False)):
        if fld in pltpu.CompilerParams.__dataclass_fields__:
            cp = dataclasses.replace(cp, **{fld: val})

    @functools.partial(
        pl.kernel,
        mesh=mesh,
        compiler_params=cp,
        out_type=jax.ShapeDtypeStruct((2, n_nodes, _H + _LANES), jnp.float32),
        scratch_types=[
            pltpu.VMEM((_C,), jnp.int32), pltpu.VMEM((_C,), jnp.int32),
            pltpu.VMEM((_C,), jnp.int32), pltpu.VMEM((_C,), jnp.int32),
            pltpu.VMEM((_C,), jnp.int32), pltpu.VMEM((_C,), jnp.int32),
            pltpu.VMEM((_C,), jnp.int32), pltpu.VMEM((_C,), jnp.int32),
            pltpu.VMEM((_C, _H + _LANES), jnp.float32),
            pltpu.VMEM((_C, _H + _LANES), jnp.float32),
            pltpu.VMEM((_C, 2 * _H), jnp.float32),
            pltpu.VMEM((_C, 2 * _H), jnp.float32),
            pltpu.VMEM((_C, _H + _LANES), jnp.float32),
            pltpu.VMEM((_C, _H + _LANES), jnp.float32),
            pltpu.VMEM_SHARED((n_nodes, _H + _LANES), jnp.float32),
            pltpu.SemaphoreType.DMA, pltpu.SemaphoreType.DMA,
            pltpu.SemaphoreType.DMA, pltpu.SemaphoreType.DMA,
            pltpu.SemaphoreType.DMA, pltpu.SemaphoreType.DMA,
        ],
    )
    def k(q_hbm, kv_hbm, s_hbm, d_hbm, a_hbm, acc_out,
          sidx0, sidx1, didx0, didx1, aidx0, aidx1, scat0, scat1,
          qbuf0, qbuf1, kvbuf0, kvbuf1, obuf0, obuf1,
          acc_s,
          semi0, semi1, semg0, semg1, sems0, sems1):
        cid = lax.axis_index("c")
        sid = lax.axis_index("s")
        wid = sid * 2 + cid

        sidx = (sidx0, sidx1)
        didx = (didx0, didx1)
        aidx = (aidx0, aidx1)
        scat = (scat0, scat1)
        qbuf = (qbuf0, qbuf1)
        kvbuf = (kvbuf0, kvbuf1)
        obuf = (obuf0, obuf1)
        semi = (semi0, semi1)
        semg = (semg0, semg1)
        sems = (sems0, sems1)

        zero16 = jnp.zeros((_LANES,), jnp.float32)

        @pl.loop(0, _C)
        def _(r):
            @pl.loop(0, _H + _LANES, step=_LANES)
            def _(j):
                obuf0[r, pl.ds(j, _LANES)] = zero16

        @pl.loop(0, rows_per_tile // _C)
        def _(t):
            r0 = sid * rows_per_tile + t * _C
            pltpu.sync_copy(obuf0.at[pl.ds(0, _C)], acc_s.at[pl.ds(r0, _C)])

        plsc.subcore_barrier()

        lanes = lax.iota(jnp.int32, _LANES)

        def idx_copies(t, b):
            base = (wid + t * _NTILES) * _C
            return (
                pltpu.make_async_copy(s_hbm.at[pl.ds(base, _C)], sidx[b], semi[b]),
                pltpu.make_async_copy(d_hbm.at[pl.ds(base, _C)], didx[b], semi[b]),
                pltpu.make_async_copy(a_hbm.at[pl.ds(base, _C)], aidx[b], semi[b]),
            )

        def gather_copies(b):
            return (
                pltpu.make_async_copy(q_hbm.at[didx[b]], qbuf[b], semg[b]),
                pltpu.make_async_copy(kv_hbm.at[sidx[b]], kvbuf[b], semg[b]),
            )

        def scatter_copies(b):
            return (
                pltpu.make_async_copy(obuf[b], acc_s.at[scat[b]], sems[b]),
            )

        def start(copies, add=False):
            for c in copies:
                c.start(add=add)

        def wait(copies):
            for c in copies:
                c.wait()

        def compute(b):
            @pl.loop(0, _C, unroll=16)
            def _(r):
                obuf[b][r, pl.ds(_H, _LANES)] = zero16

            @pl.loop(0, _C // _LANES)
            def _(g):
                rows = lanes + g * _LANES
                attrv = aidx[b][pl.ds(g * _LANES, _LANES)]

                def dot_body(j, acc):
                    jv = jnp.zeros((_LANES,), jnp.int32) + j
                    qv = plsc.load_gather(qbuf[b], [rows, jv])
                    kv_ = plsc.load_gather(kvbuf[b], [rows, jv])
                    return acc + qv * kv_

                acc = lax.fori_loop(0, _H, dot_body,
                                    jnp.zeros((_LANES,), jnp.float32),
                                    unroll=16)
                qe = plsc.load_gather(qbuf[b], [rows, attrv + _H])
                ex = jnp.exp((acc + qe) * inv_sqrt_h)
                plsc.store_scatter(obuf[b], [rows, attrv + _H], ex)

                def v_body(j, _):
                    jv = jnp.zeros((_LANES,), jnp.int32) + j
                    vv = plsc.load_gather(kvbuf[b], [rows, jv + _H])
                    plsc.store_scatter(obuf[b], [rows, jv], ex * vv)
                    return 0

                lax.fori_loop(0, _H, v_body, 0, unroll=16)

        def stage(t, b):
            wait(gather_copies(b))
            @pl.when(t + 1 < chunks_per_tile)
            def _():
                wait(idx_copies(t + 1, 1 - b))
                start(gather_copies(1 - b))
            @pl.when(t >= 2)
            def _():
                wait(scatter_copies(b))
            compute(b)

            @pl.loop(0, _C, step=_LANES)
            def _(r):
                scat[b][pl.ds(r, _LANES)] = didx[b][pl.ds(r, _LANES)]

            start(scatter_copies(b), add=True)

            @pl.when(t + 2 < chunks_per_tile)
            def _():
                start(idx_copies(t + 2, b))

        start(idx_copies(0, 0))
        start(idx_copies(1, 1))
        wait(idx_copies(0, 0))
        start(gather_copies(0))

        @pl.loop(0, chunks_per_tile, step=2)
        def _(i):
            stage(i, 0)
            stage(i + 1, 1)

        wait(scatter_copies(0))
        wait(scatter_copies(1))

        plsc.subcore_barrier()
        r0 = sid * rows_per_tile
        pltpu.sync_copy(acc_s.at[pl.ds(r0, rows_per_tile)],
                        acc_out.at[cid, pl.ds(r0, rows_per_tile)])

    return k(qT, kvT, src, dst, attr)


def kernel(x, edge_index, edge_attr, emb, W1q, b1q, W1k, b1k, W1v, b1v, W1e,
           W1s, b1s, W2q, b2q, W2k, b2k, W2v, b2v, W2e, W2s, b2s, Wo, bo):
    src = edge_index[0]
    dst = edge_index[1]
    attr = edge_attr.astype(jnp.int32)
    n = x.shape[0]
    npad = -(-n // (16 * _C)) * (16 * _C)
    x = jnp.pad(x, ((0, npad - n), (0, 0)))

    e = src.shape[0]
    epad = -(-e // (2 * _NTILES * _C)) * (2 * _NTILES * _C)
    src = jnp.pad(src, (0, epad - e))
    dst = jnp.pad(dst, (0, epad - e), constant_values=npad - 1)
    attr = jnp.pad(attr, (0, epad - e))

    bias1 = jnp.zeros((8, _H), jnp.float32)
    bias1 = bias1.at[0].set(b1q).at[1].set(b1k).at[2].set(b1v).at[3].set(b1s)
    bias2 = jnp.zeros((8, _H), jnp.float32)
    bias2 = bias2.at[0].set(b2q).at[1].set(b2k).at[2].set(b2v).at[3].set(b2s)
    biaso = jnp.zeros((8, _H), jnp.float32)
    biaso = biaso.at[0, :2].set(bo)

    qT1, kvT1, skip1, e1, e2 = _proj1(x, emb, W1q, W1k, W1v, W1s, W1e, W2e,
                                      bias1)
    accp1 = _edge_phase(qT1, kvT1, src, dst, attr)
    qT2, kvT2, skip2 = _mid(accp1, skip1, e1, e2, W2q, W2k, W2v, W2s, bias2)
    accp2 = _edge_phase(qT2, kvT2, src, dst, attr)
    return _final(accp2, skip2, e2, Wo, biaso)[:n]


# speedup vs baseline: 10.6900x; 1.8393x over previous
import dataclasses
import functools

import jax
import jax.numpy as jnp
import numpy as np
from jax import lax
from jax.experimental import pallas as pl
from jax.experimental.pallas import tpu as pltpu
from jax.experimental.pallas import tpu_sc as plsc

_H = 64
_C = 128
_NTILES = 32
_LANES = 16


def _proj1_body(x_ref, emb_ref, wq, wk, wv, ws, we1, we2, bias_ref,
                qx_ref, kv_ref, skip_ref, e1_ref, e2_ref):
    xb = x_ref[...]
    f32 = jnp.float32
    e1 = jnp.dot(emb_ref[...], we1[...], preferred_element_type=f32)
    q = jnp.dot(xb, wq[...], preferred_element_type=f32) + bias_ref[0:1, :]
    qx_ref[:, :_H] = q
    qx_ref[:, _H:] = jnp.dot(q, e1.T, preferred_element_type=f32,
                             precision=jax.lax.Precision.HIGHEST)
    kv_ref[:, :_H] = jnp.dot(xb, wk[...], preferred_element_type=f32) + bias_ref[1:2, :]
    kv_ref[:, _H:] = jnp.dot(xb, wv[...], preferred_element_type=f32) + bias_ref[2:3, :]
    skip_ref[...] = jnp.dot(xb, ws[...], preferred_element_type=f32) + bias_ref[3:4, :]

    @pl.when(pl.program_id(0) == 0)
    def _():
        e1_ref[...] = e1
        e2_ref[...] = jnp.dot(emb_ref[...], we2[...], preferred_element_type=f32)


def _combine(acc_ref, skip_ref, e_ref):
    acc = acc_ref[0] + acc_ref[1]
    s = acc[:, _H:]
    den = jnp.sum(s, axis=-1, keepdims=True)
    agg = (acc[:, :_H]
           + jnp.dot(s, e_ref[...], preferred_element_type=jnp.float32,
                     precision=jax.lax.Precision.HIGHEST))
    return jnp.maximum(agg / (den + 1e-16) + skip_ref[...], 0.0)


def _mid_body(acc_ref, skip_ref, e1_ref, e2_ref, wq, wk, wv, ws,
              bias_ref, qx_ref, kv_ref, skip2_ref):
    f32 = jnp.float32
    h = _combine(acc_ref, skip_ref, e1_ref)
    q = jnp.dot(h, wq[...], preferred_element_type=f32) + bias_ref[0:1, :]
    qx_ref[:, :_H] = q
    qx_ref[:, _H:] = jnp.dot(q, e2_ref[...].T, preferred_element_type=f32,
                             precision=jax.lax.Precision.HIGHEST)
    kv_ref[:, :_H] = jnp.dot(h, wk[...], preferred_element_type=f32) + bias_ref[1:2, :]
    kv_ref[:, _H:] = jnp.dot(h, wv[...], preferred_element_type=f32) + bias_ref[2:3, :]
    skip2_ref[...] = jnp.dot(h, ws[...], preferred_element_type=f32) + bias_ref[3:4, :]


def _final_body(acc_ref, skip_ref, e2_ref, wo, bias_ref, out_ref):
    h = _combine(acc_ref, skip_ref, e2_ref)
    out_ref[...] = (jnp.dot(h, wo[...], preferred_element_type=jnp.float32)
                    + bias_ref[0:1, :2])


def _full_spec(shape):
    return pl.BlockSpec(shape, lambda i: tuple(0 for _ in shape))


def _proj1(x, emb, wq, wk, wv, ws, we1, we2, bias):
    n, d = x.shape
    blk = 1024
    grid = (n // blk,)
    row = lambda i: (i, 0)
    return pl.pallas_call(
        _proj1_body,
        grid=grid,
        in_specs=[
            pl.BlockSpec((blk, d), row),
            _full_spec(emb.shape),
            _full_spec(wq.shape), _full_spec(wk.shape),
            _full_spec(wv.shape), _full_spec(ws.shape),
            _full_spec(we1.shape), _full_spec(we2.shape),
            _full_spec(bias.shape),
        ],
        out_specs=[
            pl.BlockSpec((blk, _H + _LANES), row),
            pl.BlockSpec((blk, 2 * _H), row),
            pl.BlockSpec((blk, _H), row),
            _full_spec((16, _H)),
            _full_spec((16, _H)),
        ],
        out_shape=[
            jax.ShapeDtypeStruct((n, _H + _LANES), jnp.float32),
            jax.ShapeDtypeStruct((n, 2 * _H), jnp.float32),
            jax.ShapeDtypeStruct((n, _H), jnp.float32),
            jax.ShapeDtypeStruct((16, _H), jnp.float32),
            jax.ShapeDtypeStruct((16, _H), jnp.float32),
        ],
    )(x, emb, wq, wk, wv, ws, we1, we2, bias)


def _mid(accp, skip, e1, e2, wq, wk, wv, ws, bias):
    n = skip.shape[0]
    blk = 1024
    grid = (n // blk,)
    row = lambda i: (i, 0)
    row3 = lambda i: (0, i, 0)
    return pl.pallas_call(
        _mid_body,
        grid=grid,
        in_specs=[
            pl.BlockSpec((2, blk, _H + _LANES), row3),
            pl.BlockSpec((blk, _H), row),
            _full_spec(e1.shape), _full_spec(e2.shape),
            _full_spec(wq.shape), _full_spec(wk.shape),
            _full_spec(wv.shape), _full_spec(ws.shape),
            _full_spec(bias.shape),
        ],
        out_specs=[
            pl.BlockSpec((blk, _H + _LANES), row),
            pl.BlockSpec((blk, 2 * _H), row),
            pl.BlockSpec((blk, _H), row),
        ],
        out_shape=[
            jax.ShapeDtypeStruct((n, _H + _LANES), jnp.float32),
            jax.ShapeDtypeStruct((n, 2 * _H), jnp.float32),
            jax.ShapeDtypeStruct((n, _H), jnp.float32),
        ],
    )(accp, skip, e1, e2, wq, wk, wv, ws, bias)


def _final(accp, skip, e2, wo, bias):
    n = skip.shape[0]
    blk = 1024
    grid = (n // blk,)
    row = lambda i: (i, 0)
    row3 = lambda i: (0, i, 0)
    return pl.pallas_call(
        _final_body,
        grid=grid,
        in_specs=[
            pl.BlockSpec((2, blk, _H + _LANES), row3),
            pl.BlockSpec((blk, _H), row),
            _full_spec(e2.shape),
            _full_spec(wo.shape),
            _full_spec(bias.shape),
        ],
        out_specs=pl.BlockSpec((blk, 2), row),
        out_shape=jax.ShapeDtypeStruct((n, 2), jnp.float32),
    )(accp, skip, e2, wo, bias)


def _edge_phase(qT, kvT, src, dst, attr):
    n_nodes = qT.shape[0]
    n_edges = src.shape[0]
    nchunk = n_edges // _C
    chunks_per_tile = nchunk // _NTILES
    rows_per_tile = n_nodes // 16
    mesh = plsc.VectorSubcoreMesh(core_axis_name="c", subcore_axis_name="s")
    inv_sqrt_h = np.float32(1.0 / np.sqrt(_H))
    cp = pltpu.CompilerParams()
    for fld, val in (("needs_layout_passes", False),
                     ("use_tc_tiling_on_sc", False)):
        if fld in pltpu.CompilerParams.__dataclass_fields__:
            cp = dataclasses.replace(cp, **{fld: val})

    @functools.partial(
        pl.kernel,
        mesh=mesh,
        compiler_params=cp,
        out_type=jax.ShapeDtypeStruct((2, n_nodes, _H + _LANES), jnp.float32),
        scratch_types=[
            pltpu.VMEM((_C,), jnp.int32), pltpu.VMEM((_C,), jnp.int32),
            pltpu.VMEM((_C,), jnp.int32), pltpu.VMEM((_C,), jnp.int32),
            pltpu.VMEM((_C,), jnp.int32), pltpu.VMEM((_C,), jnp.int32),
            pltpu.VMEM((_C,), jnp.int32), pltpu.VMEM((_C,), jnp.int32),
            pltpu.VMEM((_C, _H + _LANES), jnp.float32),
            pltpu.VMEM((_C, _H + _LANES), jnp.float32),
            pltpu.VMEM((_C, 2 * _H), jnp.float32),
            pltpu.VMEM((_C, 2 * _H), jnp.float32),
            pltpu.VMEM((_C, _H + _LANES), jnp.float32),
            pltpu.VMEM((_C, _H + _LANES), jnp.float32),
            pltpu.VMEM((_LANES, _LANES), jnp.float32),
            pltpu.VMEM_SHARED((n_nodes, _H + _LANES), jnp.float32),
            pltpu.SemaphoreType.DMA, pltpu.SemaphoreType.DMA,
            pltpu.SemaphoreType.DMA, pltpu.SemaphoreType.DMA,
            pltpu.SemaphoreType.DMA, pltpu.SemaphoreType.DMA,
        ],
    )
    def k(q_hbm, kv_hbm, s_hbm, d_hbm, a_hbm, acc_out,
          sidx0, sidx1, didx0, didx1, aidx0, aidx1, scat0, scat1,
          qbuf0, qbuf1, kvbuf0, kvbuf1, obuf0, obuf1, dots,
          acc_s,
          semi0, semi1, semg0, semg1, sems0, sems1):
        cid = lax.axis_index("c")
        sid = lax.axis_index("s")
        wid = sid * 2 + cid

        sidx = (sidx0, sidx1)
        didx = (didx0, didx1)
        aidx = (aidx0, aidx1)
        scat = (scat0, scat1)
        qbuf = (qbuf0, qbuf1)
        kvbuf = (kvbuf0, kvbuf1)
        obuf = (obuf0, obuf1)
        semi = (semi0, semi1)
        semg = (semg0, semg1)
        sems = (sems0, sems1)

        zero16 = jnp.zeros((_LANES,), jnp.float32)

        @pl.loop(0, _C)
        def _(r):
            @pl.loop(0, _H + _LANES, step=_LANES)
            def _(j):
                obuf0[r, pl.ds(j, _LANES)] = zero16

        @pl.loop(0, rows_per_tile // _C)
        def _(t):
            r0 = sid * rows_per_tile + t * _C
            pltpu.sync_copy(obuf0.at[pl.ds(0, _C)], acc_s.at[pl.ds(r0, _C)])

        plsc.subcore_barrier()

        lanes = lax.iota(jnp.int32, _LANES)

        def idx_copies(t, b):
            base = (wid + t * _NTILES) * _C
            return (
                pltpu.make_async_copy(s_hbm.at[pl.ds(base, _C)], sidx[b], semi[b]),
                pltpu.make_async_copy(d_hbm.at[pl.ds(base, _C)], didx[b], semi[b]),
                pltpu.make_async_copy(a_hbm.at[pl.ds(base, _C)], aidx[b], semi[b]),
            )

        def gather_copies(b):
            return (
                pltpu.make_async_copy(q_hbm.at[didx[b]], qbuf[b], semg[b]),
                pltpu.make_async_copy(kv_hbm.at[sidx[b]], kvbuf[b], semg[b]),
            )

        def scatter_copies(b):
            return (
                pltpu.make_async_copy(obuf[b], acc_s.at[scat[b]], sems[b]),
            )

        def start(copies, add=False):
            for c in copies:
                c.start(add=add)

        def wait(copies):
            for c in copies:
                c.wait()

        def compute(b):
            @pl.loop(0, _C // _LANES)
            def _(g):
                r0 = g * _LANES
                rows = lanes + r0
                attrv = aidx[b][pl.ds(r0, _LANES)]

                for e in range(_LANES):
                    r = r0 + e
                    acc = (qbuf[b][r, pl.ds(0, _LANES)]
                           * kvbuf[b][r, pl.ds(0, _LANES)])
                    for s in range(1, _H // _LANES):
                        acc = acc + (qbuf[b][r, pl.ds(s * _LANES, _LANES)]
                                     * kvbuf[b][r, pl.ds(s * _LANES, _LANES)])
                    dots[e, pl.ds(0, _LANES)] = acc

                alpha = jnp.zeros((_LANES,), jnp.float32)
                for j in range(_LANES):
                    jv = jnp.zeros((_LANES,), jnp.int32) + j
                    alpha = alpha + plsc.load_gather(dots, [lanes, jv])

                qe = plsc.load_gather(qbuf[b], [rows, attrv + _H])
                ex = jnp.exp((alpha + qe) * inv_sqrt_h)

                for e in range(_LANES):
                    r = r0 + e
                    exr = jnp.full((_LANES,), ex[e])
                    obuf[b][r, pl.ds(_H, _LANES)] = jnp.where(
                        lanes == attrv[e], exr, 0.0)
                    for s in range(_H // _LANES):
                        obuf[b][r, pl.ds(s * _LANES, _LANES)] = (
                            exr * kvbuf[b][r, pl.ds(_H + s * _LANES, _LANES)])

        def stage(t, b):
            wait(gather_copies(b))
            @pl.when(t + 1 < chunks_per_tile)
            def _():
                wait(idx_copies(t + 1, 1 - b))
                start(gather_copies(1 - b))
            @pl.when(t >= 2)
            def _():
                wait(scatter_copies(b))
            compute(b)

            @pl.loop(0, _C, step=_LANES)
            def _(r):
                scat[b][pl.ds(r, _LANES)] = didx[b][pl.ds(r, _LANES)]

            start(scatter_copies(b), add=True)

            @pl.when(t + 2 < chunks_per_tile)
            def _():
                start(idx_copies(t + 2, b))

        start(idx_copies(0, 0))
        start(idx_copies(1, 1))
        wait(idx_copies(0, 0))
        start(gather_copies(0))

        @pl.loop(0, chunks_per_tile, step=2)
        def _(i):
            stage(i, 0)
            stage(i + 1, 1)

        wait(scatter_copies(0))
        wait(scatter_copies(1))

        plsc.subcore_barrier()
        r0 = sid * rows_per_tile
        pltpu.sync_copy(acc_s.at[pl.ds(r0, rows_per_tile)],
                        acc_out.at[cid, pl.ds(r0, rows_per_tile)])

    return k(qT, kvT, src, dst, attr)


def kernel(x, edge_index, edge_attr, emb, W1q, b1q, W1k, b1k, W1v, b1v, W1e,
           W1s, b1s, W2q, b2q, W2k, b2k, W2v, b2v, W2e, W2s, b2s, Wo, bo):
    src = edge_index[0]
    dst = edge_index[1]
    attr = edge_attr.astype(jnp.int32)
    n = x.shape[0]
    npad = -(-n // (16 * _C)) * (16 * _C)
    x = jnp.pad(x, ((0, npad - n), (0, 0)))

    e = src.shape[0]
    epad = -(-e // (2 * _NTILES * _C)) * (2 * _NTILES * _C)
    src = jnp.pad(src, (0, epad - e))
    dst = jnp.pad(dst, (0, epad - e), constant_values=npad - 1)
    attr = jnp.pad(attr, (0, epad - e))

    bias1 = jnp.zeros((8, _H), jnp.float32)
    bias1 = bias1.at[0].set(b1q).at[1].set(b1k).at[2].set(b1v).at[3].set(b1s)
    bias2 = jnp.zeros((8, _H), jnp.float32)
    bias2 = bias2.at[0].set(b2q).at[1].set(b2k).at[2].set(b2v).at[3].set(b2s)
    biaso = jnp.zeros((8, _H), jnp.float32)
    biaso = biaso.at[0, :2].set(bo)

    qT1, kvT1, skip1, e1, e2 = _proj1(x, emb, W1q, W1k, W1v, W1s, W1e, W2e,
                                      bias1)
    accp1 = _edge_phase(qT1, kvT1, src, dst, attr)
    qT2, kvT2, skip2 = _mid(accp1, skip1, e1, e2, W2q, W2k, W2v, W2s, bias2)
    accp2 = _edge_phase(qT2, kvT2, src, dst, attr)
    return _final(accp2, skip2, e2, Wo, biaso)[:n]
